# baseline (speedup 1.0000x reference)
import functools

import jax
import jax.numpy as jnp
from jax import lax
from jax.experimental import pallas as pl
from jax.experimental.pallas import tpu as pltpu
from jax.experimental.pallas import tpu_sc as plsc

VOCAB = 1000000
DIM = 64
BATCH = 16384
HIST = 50

NC = 2
NS = 16
NW = NC * NS
N = BATCH * HIST
PER_W = N // NW
C = 128
N_CHUNKS = PER_W // C


def _body(idx_hbm, qw_hbm, sc_hbm, out_hbm, idx_v, rows_v, s_v, out_v,
          sem_r, sem_s):
    wid = lax.axis_index("s") * NC + lax.axis_index("c")
    base = wid * PER_W

    @pl.loop(0, N_CHUNKS)
    def _chunk(g):
        off = base + g * C
        pltpu.sync_copy(idx_hbm.at[pl.ds(off, C)], idx_v)
        cp_r = pltpu.async_copy(qw_hbm.at[idx_v], rows_v, sem_r)
        cp_s = pltpu.async_copy(sc_hbm.at[idx_v], s_v, sem_s)
        cp_r.wait()
        cp_s.wait()

        @pl.loop(0, C // 16)
        def _row16(i16):
            s16 = s_v[pl.ds(i16 * 16, 16)]
            for r in range(16):
                i = i16 * 16 + r
                s = s16[r]
                for j in range(DIM // 16):
                    q = rows_v[i, pl.ds(j * 16, 16)]
                    out_v[i, pl.ds(j * 16, 16)] = (
                        q.astype(jnp.float32) - 8.0) * s

        pltpu.sync_copy(out_v, out_hbm.at[pl.ds(off, C)])


@functools.partial(
    pl.kernel,
    out_type=jax.ShapeDtypeStruct((N, DIM), jnp.float32),
    mesh=plsc.VectorSubcoreMesh(
        core_axis_name="c", subcore_axis_name="s",
        num_cores=NC, num_subcores=NS),
    scratch_types=[
        pltpu.VMEM((C,), jnp.int32),
        pltpu.VMEM((C, DIM), jnp.int32),
        pltpu.VMEM((C,), jnp.float32),
        pltpu.VMEM((C, DIM), jnp.float32),
        pltpu.SemaphoreType.DMA,
        pltpu.SemaphoreType.DMA,
    ],
    compiler_params=pltpu.CompilerParams(use_tc_tiling_on_sc=False),
)
def _sc_lookup(idx_hbm, qw_hbm, sc_hbm, out_hbm, idx_v, rows_v, s_v, out_v,
               sem_r, sem_s):
    _body(idx_hbm, qw_hbm, sc_hbm, out_hbm, idx_v, rows_v, s_v, out_v,
          sem_r, sem_s)


def kernel(indices, qweights, scales):
    flat_idx = indices.reshape(N)
    out = _sc_lookup(flat_idx, qweights, scales)
    return out.reshape(BATCH, HIST, DIM)


# speedup vs baseline: 1.2374x; 1.2374x over previous
import functools

import jax
import jax.numpy as jnp
from jax import lax
from jax.experimental import pallas as pl
from jax.experimental.pallas import tpu as pltpu
from jax.experimental.pallas import tpu_sc as plsc

VOCAB = 1000000
DIM = 64
BATCH = 16384
HIST = 50

NC = 2
NS = 16
NW = NC * NS
N = BATCH * HIST
PER_W = N // NW
CH = 128
CHUNKS = PER_W // CH
NBUF = 2


def _dequant_chunk(rows_ref, s_ref, out_ref):
    @pl.loop(0, CH // 16)
    def _row16(i16):
        s16 = s_ref[pl.ds(i16 * 16, 16)]
        for r in range(16):
            i = i16 * 16 + r
            s = s16[r]
            for j in range(DIM // 16):
                q = rows_ref[i, pl.ds(j * 16, 16)]
                out_ref[i, pl.ds(j * 16, 16)] = (
                    q.astype(jnp.float32) - 8.0) * s


@functools.partial(
    pl.kernel,
    out_type=jax.ShapeDtypeStruct((N, DIM), jnp.float32),
    mesh=plsc.VectorSubcoreMesh(
        core_axis_name="c", subcore_axis_name="s",
        num_cores=NC, num_subcores=NS),
    scratch_types=[
        pltpu.VMEM((CHUNKS, CH), jnp.int32),
        pltpu.VMEM((CH, DIM), jnp.int32),
        pltpu.VMEM((CH, DIM), jnp.int32),
        pltpu.VMEM((CH,), jnp.float32),
        pltpu.VMEM((CH,), jnp.float32),
        pltpu.VMEM((CH, DIM), jnp.float32),
        pltpu.VMEM((CH, DIM), jnp.float32),
        pltpu.SemaphoreType.DMA,
        pltpu.SemaphoreType.DMA,
        pltpu.SemaphoreType.DMA,
        pltpu.SemaphoreType.DMA,
        pltpu.SemaphoreType.DMA,
        pltpu.SemaphoreType.DMA,
    ],
    compiler_params=pltpu.CompilerParams(use_tc_tiling_on_sc=False),
)
def _sc_lookup(idx_hbm, qw_hbm, sc_hbm, out_hbm, idx_v,
               rows0, rows1, s0, s1, o0, o1,
               sem_r0, sem_r1, sem_s0, sem_s1, sem_o0, sem_o1):
    rows = (rows0, rows1)
    sv = (s0, s1)
    ov = (o0, o1)
    sem_r = (sem_r0, sem_r1)
    sem_s = (sem_s0, sem_s1)
    sem_o = (sem_o0, sem_o1)

    wid = lax.axis_index("s") * NC + lax.axis_index("c")
    base = wid * PER_W

    pltpu.sync_copy(idx_hbm.at[wid], idx_v)

    for b in range(NBUF):
        pltpu.async_copy(qw_hbm.at[idx_v.at[b]], rows[b], sem_r[b])
        pltpu.async_copy(sc_hbm.at[idx_v.at[b]], sv[b], sem_s[b])

    @pl.loop(0, CHUNKS, step=NBUF)
    def _g(g0):
        for b in range(NBUF):
            g = g0 + b
            pltpu.make_async_copy(qw_hbm.at[idx_v.at[g]], rows[b],
                                  sem_r[b]).wait()
            pltpu.make_async_copy(sc_hbm.at[idx_v.at[g]], sv[b],
                                  sem_s[b]).wait()
            @pl.when(g >= NBUF)
            def _():
                pltpu.make_async_copy(
                    ov[b], out_hbm.at[pl.ds(base, CH)], sem_o[b]).wait()

            _dequant_chunk(rows[b], sv[b], ov[b])

            pltpu.async_copy(ov[b], out_hbm.at[pl.ds(base + g * CH, CH)],
                             sem_o[b])
            ng = g + NBUF

            @pl.when(ng < CHUNKS)
            def _():
                pltpu.async_copy(qw_hbm.at[idx_v.at[ng]], rows[b], sem_r[b])
                pltpu.async_copy(sc_hbm.at[idx_v.at[ng]], sv[b], sem_s[b])

    for b in range(NBUF):
        pltpu.make_async_copy(ov[b], out_hbm.at[pl.ds(base, CH)],
                              sem_o[b]).wait()


def kernel(indices, qweights, scales):
    idx3 = indices.reshape(NW, CHUNKS, CH)
    out = _sc_lookup(idx3, qweights, scales)
    return out.reshape(BATCH, HIST, DIM)
